# Initial kernel scaffold; baseline (speedup 1.0000x reference)
import jax
import jax.numpy as jnp
from jax.experimental import pallas as pl


def kernel(tokens, embed_table):
    raise NotImplementedError("write your pallas kernel here")


# speedup vs baseline: 2.6488x; 2.6488x over previous
import jax
import jax.numpy as jnp
from jax import lax
from jax.experimental import pallas as pl
from jax.experimental.pallas import tpu as pltpu
from jax.experimental.pallas import tpu_sc as plsc

PAD = 1
B = 4
S = 8192
D = 64
NCORES = 2
NSUB = 16
NW = NCORES * NSUB
CHUNK = (B * S) // NW
CPR = S // CHUNK
ROWS_PER_CORE = B // NCORES
NVREG = CHUNK // 16
IDX_W = 128
NGATHER = CHUNK // IDX_W


def _body(tok_hbm, tab_hbm, out_hbm, tok_v, pos_v, rows_v, tot_v, grp_v,
          shared_tot, sem):
    c = lax.axis_index("c")
    s = lax.axis_index("s")
    row = c * ROWS_PER_CORE + s // CPR
    cir = s % CPR
    base = (row * CPR + cir) * CHUNK

    pltpu.sync_copy(tok_hbm.at[pl.ds(base, CHUNK)], tok_v)

    pad_v = jnp.broadcast_to(jnp.int32(PAD), (16,))

    acc = jnp.zeros((16,), jnp.int32)
    for i in range(NVREG):
        t = tok_v[pl.ds(i * 16, 16)]
        acc = acc + (t != pad_v).astype(jnp.int32)
    total = jnp.sum(acc)
    tot_v[...] = jnp.broadcast_to(total, (16,))
    pltpu.sync_copy(tot_v, shared_tot.at[s])
    plsc.subcore_barrier()

    grp_base = (s // CPR) * CPR
    pltpu.sync_copy(shared_tot.at[pl.ds(grp_base, CPR)], grp_v)
    offset = jnp.int32(0)
    for j in range(CPR):
        tj = jnp.max(grp_v[j, :])
        offset = offset + jnp.where(j < cir, tj, jnp.int32(0))

    run = offset
    for i in range(NVREG):
        t = tok_v[pl.ds(i * 16, 16)]
        m = (t != pad_v).astype(jnp.int32)
        cum = plsc.cumsum(m) + jnp.broadcast_to(run, (16,))
        pos = cum * m + pad_v
        pos_v[i // (IDX_W // 16), pl.ds((i % (IDX_W // 16)) * 16, 16)] = pos
        run = run + jnp.sum(m)

    handles = [
        pltpu.async_copy(tab_hbm.at[pos_v.at[j]],
                         rows_v.at[pl.ds(j * IDX_W, IDX_W)], sem)
        for j in range(NGATHER)
    ]
    for h in handles:
        h.wait()

    pltpu.sync_copy(rows_v, out_hbm.at[pl.ds(base, CHUNK)])


def kernel(tokens, embed_table):
    tok = tokens.reshape(B * S).astype(jnp.int32)
    mesh = plsc.VectorSubcoreMesh(core_axis_name="c", subcore_axis_name="s")
    run_k = pl.kernel(
        _body,
        mesh=mesh,
        compiler_params=pltpu.CompilerParams(
            use_tc_tiling_on_sc=False, needs_layout_passes=False),
        out_type=jax.ShapeDtypeStruct((B * S, D), jnp.float32),
        scratch_types=[
            pltpu.VMEM((CHUNK,), jnp.int32),
            pltpu.VMEM((NGATHER, IDX_W), jnp.int32),
            pltpu.VMEM((CHUNK, D), jnp.float32),
            pltpu.VMEM((16,), jnp.int32),
            pltpu.VMEM((CPR, 16), jnp.int32),
            pltpu.VMEM_SHARED((NSUB, 16), jnp.int32),
            pltpu.SemaphoreType.DMA,
        ],
    )
    out = run_k(tok, embed_table)
    return out.reshape(B, S, D)
